# Initial kernel scaffold; baseline (speedup 1.0000x reference)
import jax
import jax.numpy as jnp
from jax.experimental import pallas as pl


def kernel(input, adj, G, W1, b1, Wc, W2, b2):
    raise NotImplementedError("write your pallas kernel here")


# speedup vs baseline: 1.0806x; 1.0806x over previous
import math
import functools

import jax
import jax.numpy as jnp
from jax.experimental import pallas as pl
from jax.experimental.pallas import tpu as pltpu

LAMDA = 0.5
ALPHA = 0.1


def _body(x_ref, g_ref, w1_ref, b1_ref, wc_ref, w2_ref, b2_ref,
          out_ref, h0_ref, h_ref, *, bm, nlayers, thetas):
    l = pl.program_id(0)
    i = pl.program_id(1)

    @pl.when((l == 0) & (i == 0))
    def _init():
        h0 = jnp.maximum(
            jnp.dot(x_ref[...], w1_ref[...],
                    preferred_element_type=jnp.float32) + b1_ref[...], 0.0)
        h0_ref[...] = h0
        h_ref[1] = h0

    h_cur = h_ref[(l + 1) % 2]
    hi = jnp.dot(g_ref[...], h_cur, preferred_element_type=jnp.float32)
    h0_blk = h0_ref[pl.ds(i * bm, bm), :]
    support = (1.0 - ALPHA) * hi + ALPHA * h0_blk

    theta = thetas[0]
    for k in range(1, nlayers):
        theta = jnp.where(l == k, thetas[k], theta)

    wc = wc_ref[0]
    hnew = jnp.maximum(
        theta * jnp.dot(support, wc, preferred_element_type=jnp.float32)
        + (1.0 - theta) * support, 0.0)
    h_ref[l % 2, pl.ds(i * bm, bm), :] = hnew

    @pl.when(l == nlayers - 1)
    def _proj():
        out_ref[...] = (jnp.dot(hnew, w2_ref[...],
                                preferred_element_type=jnp.float32)
                        + b2_ref[...])


@jax.jit
def kernel(input, adj, G, W1, b1, Wc, W2, b2):
    del adj
    n, nfeat = input.shape
    nhidden = W1.shape[1]
    nclass = W2.shape[1]
    nlayers = Wc.shape[0]
    bm = 400 if n % 400 == 0 else n
    nblocks = n // bm
    thetas = tuple(math.log(LAMDA / (k + 1) + 1.0) for k in range(nlayers))

    body = functools.partial(_body, bm=bm, nlayers=nlayers, thetas=thetas)
    out = pl.pallas_call(
        body,
        grid=(nlayers, nblocks),
        in_specs=[
            pl.BlockSpec((n, nfeat), lambda l, i: (0, 0)),
            pl.BlockSpec((bm, n), lambda l, i: (i, 0)),
            pl.BlockSpec((nfeat, nhidden), lambda l, i: (0, 0)),
            pl.BlockSpec((1, nhidden), lambda l, i: (0, 0)),
            pl.BlockSpec((1, nhidden, nhidden), lambda l, i: (l, 0, 0)),
            pl.BlockSpec((nhidden, nclass), lambda l, i: (0, 0)),
            pl.BlockSpec((1, nclass), lambda l, i: (0, 0)),
        ],
        out_specs=pl.BlockSpec((bm, nclass), lambda l, i: (i, 0)),
        out_shape=jax.ShapeDtypeStruct((n, nclass), jnp.float32),
        scratch_shapes=[
            pltpu.VMEM((n, nhidden), jnp.float32),
            pltpu.VMEM((2, n, nhidden), jnp.float32),
        ],
    )(input, G, W1, b1.reshape(1, -1), Wc, W2, b2.reshape(1, -1))
    return out


# speedup vs baseline: 1.2605x; 1.1664x over previous
import math
import functools

import jax
import jax.numpy as jnp
from jax.experimental import pallas as pl
from jax.experimental.pallas import tpu as pltpu

LAMDA = 0.5
ALPHA = 0.1


def _stage1_body(x_ref, g_ref, w1_ref, b1_ref, wc0_ref,
                 gbf_ref, h1_ref, h0_ref, *, bm, theta0):
    i = pl.program_id(0)

    @pl.when(i == 0)
    def _init():
        h0_ref[...] = jnp.maximum(
            jnp.dot(x_ref[...], w1_ref[...],
                    preferred_element_type=jnp.float32) + b1_ref[...], 0.0)

    g = g_ref[...]
    gbf_ref[...] = g.astype(jnp.bfloat16)
    hi = jnp.dot(g, h0_ref[...], preferred_element_type=jnp.float32)
    support = (1.0 - ALPHA) * hi + ALPHA * h0_ref[pl.ds(i * bm, bm), :]
    h1_ref[...] = jnp.maximum(
        theta0 * jnp.dot(support, wc0_ref[0],
                         preferred_element_type=jnp.float32)
        + (1.0 - theta0) * support, 0.0)


def _stage2_body(gbf_ref, h1_ref, h0_ref, wc_ref, w2_ref, b2_ref,
                 out_ref, hb_ref, *, bm, nrest, thetas):
    l = pl.program_id(0)
    i = pl.program_id(1)

    @pl.when((l == 0) & (i == 0))
    def _init():
        hb_ref[1] = h1_ref[...].astype(jnp.bfloat16)

    h_cur = hb_ref[(l + 1) % 2]
    hi = jnp.dot(gbf_ref[...], h_cur, preferred_element_type=jnp.float32)
    support = (1.0 - ALPHA) * hi + ALPHA * h0_ref[pl.ds(i * bm, bm), :]

    theta = thetas[0]
    for k in range(1, nrest):
        theta = jnp.where(l == k, thetas[k], theta)

    hnew = jnp.maximum(
        theta * jnp.dot(support, wc_ref[0],
                        preferred_element_type=jnp.float32)
        + (1.0 - theta) * support, 0.0)
    hb_ref[l % 2, pl.ds(i * bm, bm), :] = hnew.astype(jnp.bfloat16)

    @pl.when(l == nrest - 1)
    def _proj():
        out_ref[...] = (jnp.dot(hnew, w2_ref[...],
                                preferred_element_type=jnp.float32)
                        + b2_ref[...])


@jax.jit
def kernel(input, adj, G, W1, b1, Wc, W2, b2):
    del adj
    n, nfeat = input.shape
    nhidden = W1.shape[1]
    nclass = W2.shape[1]
    nlayers = Wc.shape[0]
    bm1 = 200 if n % 200 == 0 else n
    bm2 = 400 if n % 400 == 0 else n
    thetas = tuple(math.log(LAMDA / (k + 1) + 1.0) for k in range(nlayers))

    s1 = functools.partial(_stage1_body, bm=bm1, theta0=thetas[0])
    gbf, h1, h0 = pl.pallas_call(
        s1,
        grid=(n // bm1,),
        in_specs=[
            pl.BlockSpec((n, nfeat), lambda i: (0, 0)),
            pl.BlockSpec((bm1, n), lambda i: (i, 0)),
            pl.BlockSpec((nfeat, nhidden), lambda i: (0, 0)),
            pl.BlockSpec((1, nhidden), lambda i: (0, 0)),
            pl.BlockSpec((1, nhidden, nhidden), lambda i: (0, 0, 0)),
        ],
        out_specs=[
            pl.BlockSpec((bm1, n), lambda i: (i, 0)),
            pl.BlockSpec((bm1, nhidden), lambda i: (i, 0)),
            pl.BlockSpec((n, nhidden), lambda i: (0, 0)),
        ],
        out_shape=[
            jax.ShapeDtypeStruct((n, n), jnp.bfloat16),
            jax.ShapeDtypeStruct((n, nhidden), jnp.float32),
            jax.ShapeDtypeStruct((n, nhidden), jnp.float32),
        ],
    )(input, G, W1, b1.reshape(1, -1), Wc)

    nrest = nlayers - 1
    s2 = functools.partial(_stage2_body, bm=bm2, nrest=nrest,
                           thetas=thetas[1:])
    out = pl.pallas_call(
        s2,
        grid=(nrest, n // bm2),
        in_specs=[
            pl.BlockSpec((bm2, n), lambda l, i: (i, 0)),
            pl.BlockSpec((n, nhidden), lambda l, i: (0, 0)),
            pl.BlockSpec((n, nhidden), lambda l, i: (0, 0)),
            pl.BlockSpec((1, nhidden, nhidden), lambda l, i: (l + 1, 0, 0)),
            pl.BlockSpec((nhidden, nclass), lambda l, i: (0, 0)),
            pl.BlockSpec((1, nclass), lambda l, i: (0, 0)),
        ],
        out_specs=pl.BlockSpec((bm2, nclass), lambda l, i: (i, 0)),
        out_shape=jax.ShapeDtypeStruct((n, nclass), jnp.float32),
        scratch_shapes=[
            pltpu.VMEM((2, n, nhidden), jnp.bfloat16),
        ],
    )(gbf, h1, h0, Wc, W2, b2.reshape(1, -1))
    return out


# speedup vs baseline: 1.3074x; 1.0372x over previous
import math
import functools

import jax
import jax.numpy as jnp
from jax.experimental import pallas as pl
from jax.experimental.pallas import tpu as pltpu

LAMDA = 0.5
ALPHA = 0.1


def _stage1_body(x_ref, g_ref, w1_ref, b1_ref, wc0_ref,
                 gbf_ref, h1_ref, h0_ref, *, bm, theta0):
    i = pl.program_id(0)

    @pl.when(i == 0)
    def _init():
        h0_ref[...] = jnp.maximum(
            jnp.dot(x_ref[...], w1_ref[...],
                    preferred_element_type=jnp.float32) + b1_ref[...], 0.0)

    g = g_ref[...]
    gbf_ref[...] = g.astype(jnp.bfloat16)
    hi = jnp.dot(g, h0_ref[...], preferred_element_type=jnp.float32)
    support = (1.0 - ALPHA) * hi + ALPHA * h0_ref[pl.ds(i * bm, bm), :]
    h1_ref[...] = jnp.maximum(
        theta0 * jnp.dot(support, wc0_ref[0],
                         preferred_element_type=jnp.float32)
        + (1.0 - theta0) * support, 0.0)


def _stage2_body(gbf_ref, h1_ref, h0_ref, wc_ref, w2_ref, b2_ref,
                 out_ref, hb_ref, *, bm, nrest, thetas):
    l = pl.program_id(0)
    i = pl.program_id(1)

    @pl.when((l == 0) & (i == 0))
    def _init():
        hb_ref[1] = h1_ref[...].astype(jnp.bfloat16)

    h_cur = hb_ref[(l + 1) % 2]
    hi = jnp.dot(gbf_ref[...], h_cur, preferred_element_type=jnp.float32)
    support = (1.0 - ALPHA) * hi + ALPHA * h0_ref[pl.ds(i * bm, bm), :]

    theta = thetas[0]
    for k in range(1, nrest):
        theta = jnp.where(l == k, thetas[k], theta)

    hnew = jnp.maximum(
        theta * jnp.dot(support, wc_ref[0],
                        preferred_element_type=jnp.float32)
        + (1.0 - theta) * support, 0.0)
    hb_ref[l % 2, pl.ds(i * bm, bm), :] = hnew.astype(jnp.bfloat16)

    @pl.when(l == nrest - 1)
    def _proj():
        out_ref[...] = (jnp.dot(hnew, w2_ref[...],
                                preferred_element_type=jnp.float32)
                        + b2_ref[...])


@jax.jit
def kernel(input, adj, G, W1, b1, Wc, W2, b2):
    del adj
    n, nfeat = input.shape
    nhidden = W1.shape[1]
    nclass = W2.shape[1]
    nlayers = Wc.shape[0]
    bm1 = 200 if n % 200 == 0 else n
    bm2 = 1000 if n % 1000 == 0 else n
    thetas = tuple(math.log(LAMDA / (k + 1) + 1.0) for k in range(nlayers))

    s1 = functools.partial(_stage1_body, bm=bm1, theta0=thetas[0])
    gbf, h1, h0 = pl.pallas_call(
        s1,
        grid=(n // bm1,),
        in_specs=[
            pl.BlockSpec((n, nfeat), lambda i: (0, 0)),
            pl.BlockSpec((bm1, n), lambda i: (i, 0)),
            pl.BlockSpec((nfeat, nhidden), lambda i: (0, 0)),
            pl.BlockSpec((1, nhidden), lambda i: (0, 0)),
            pl.BlockSpec((1, nhidden, nhidden), lambda i: (0, 0, 0)),
        ],
        out_specs=[
            pl.BlockSpec((bm1, n), lambda i: (i, 0)),
            pl.BlockSpec((bm1, nhidden), lambda i: (i, 0)),
            pl.BlockSpec((n, nhidden), lambda i: (0, 0)),
        ],
        out_shape=[
            jax.ShapeDtypeStruct((n, n), jnp.bfloat16),
            jax.ShapeDtypeStruct((n, nhidden), jnp.float32),
            jax.ShapeDtypeStruct((n, nhidden), jnp.float32),
        ],
    )(input, G, W1, b1.reshape(1, -1), Wc)

    nrest = nlayers - 1
    s2 = functools.partial(_stage2_body, bm=bm2, nrest=nrest,
                           thetas=thetas[1:])
    out = pl.pallas_call(
        s2,
        grid=(nrest, n // bm2),
        in_specs=[
            pl.BlockSpec((bm2, n), lambda l, i: (i, 0)),
            pl.BlockSpec((n, nhidden), lambda l, i: (0, 0)),
            pl.BlockSpec((n, nhidden), lambda l, i: (0, 0)),
            pl.BlockSpec((1, nhidden, nhidden), lambda l, i: (l + 1, 0, 0)),
            pl.BlockSpec((nhidden, nclass), lambda l, i: (0, 0)),
            pl.BlockSpec((1, nclass), lambda l, i: (0, 0)),
        ],
        out_specs=pl.BlockSpec((bm2, nclass), lambda l, i: (i, 0)),
        out_shape=jax.ShapeDtypeStruct((n, nclass), jnp.float32),
        scratch_shapes=[
            pltpu.VMEM((2, n, nhidden), jnp.bfloat16),
        ],
    )(gbf, h1, h0, Wc, W2, b2.reshape(1, -1))
    return out


# speedup vs baseline: 1.3275x; 1.0154x over previous
import math
import functools

import jax
import jax.numpy as jnp
from jax.experimental import pallas as pl
from jax.experimental.pallas import tpu as pltpu

LAMDA = 0.5
ALPHA = 0.1


def _stage1_body(x_ref, g_ref, w1_ref, b1_ref, wc0_ref,
                 gbf_ref, h1_ref, h0_ref, *, bm, theta0):
    i = pl.program_id(0)

    @pl.when(i == 0)
    def _init():
        h0_ref[...] = jnp.maximum(
            jnp.dot(x_ref[...], w1_ref[...],
                    preferred_element_type=jnp.float32) + b1_ref[...], 0.0)

    g = g_ref[...]
    gbf_ref[...] = g.astype(jnp.bfloat16)
    hi = jnp.dot(g, h0_ref[...], preferred_element_type=jnp.float32)
    support = (1.0 - ALPHA) * hi + ALPHA * h0_ref[pl.ds(i * bm, bm), :]
    h1_ref[...] = jnp.maximum(
        theta0 * jnp.dot(support, wc0_ref[0],
                         preferred_element_type=jnp.float32)
        + (1.0 - theta0) * support, 0.0)


def _stage2_body(gbf_ref, h1_ref, h0_ref, wc_ref, w2_ref, b2_ref,
                 out_ref, hb_ref, *, bm, nrest, thetas):
    l = pl.program_id(0)
    i = pl.program_id(1)

    @pl.when((l == 0) & (i == 0))
    def _init():
        hb_ref[1] = h1_ref[...].astype(jnp.bfloat16)

    h_cur = hb_ref[(l + 1) % 2]
    hi = jnp.dot(gbf_ref[...], h_cur, preferred_element_type=jnp.float32)
    support = (1.0 - ALPHA) * hi + ALPHA * h0_ref[pl.ds(i * bm, bm), :]

    theta = thetas[0]
    for k in range(1, nrest):
        theta = jnp.where(l == k, thetas[k], theta)

    hnew = jnp.maximum(
        theta * jnp.dot(support, wc_ref[0],
                        preferred_element_type=jnp.float32)
        + (1.0 - theta) * support, 0.0)
    hb_ref[l % 2, pl.ds(i * bm, bm), :] = hnew.astype(jnp.bfloat16)

    @pl.when(l == nrest - 1)
    def _proj():
        out_ref[...] = (jnp.dot(hnew, w2_ref[...],
                                preferred_element_type=jnp.float32)
                        + b2_ref[...])


@jax.jit
def kernel(input, adj, G, W1, b1, Wc, W2, b2):
    del adj
    n, nfeat = input.shape
    nhidden = W1.shape[1]
    nclass = W2.shape[1]
    nlayers = Wc.shape[0]
    bm1 = 400 if n % 400 == 0 else n
    bm2 = 1000 if n % 1000 == 0 else n
    thetas = tuple(math.log(LAMDA / (k + 1) + 1.0) for k in range(nlayers))

    s1 = functools.partial(_stage1_body, bm=bm1, theta0=thetas[0])
    gbf, h1, h0 = pl.pallas_call(
        s1,
        grid=(n // bm1,),
        in_specs=[
            pl.BlockSpec((n, nfeat), lambda i: (0, 0)),
            pl.BlockSpec((bm1, n), lambda i: (i, 0)),
            pl.BlockSpec((nfeat, nhidden), lambda i: (0, 0)),
            pl.BlockSpec((1, nhidden), lambda i: (0, 0)),
            pl.BlockSpec((1, nhidden, nhidden), lambda i: (0, 0, 0)),
        ],
        out_specs=[
            pl.BlockSpec((bm1, n), lambda i: (i, 0)),
            pl.BlockSpec((bm1, nhidden), lambda i: (i, 0)),
            pl.BlockSpec((n, nhidden), lambda i: (0, 0)),
        ],
        out_shape=[
            jax.ShapeDtypeStruct((n, n), jnp.bfloat16),
            jax.ShapeDtypeStruct((n, nhidden), jnp.float32),
            jax.ShapeDtypeStruct((n, nhidden), jnp.float32),
        ],
    )(input, G, W1, b1.reshape(1, -1), Wc)

    nrest = nlayers - 1
    s2 = functools.partial(_stage2_body, bm=bm2, nrest=nrest,
                           thetas=thetas[1:])
    out = pl.pallas_call(
        s2,
        grid=(nrest, n // bm2),
        in_specs=[
            pl.BlockSpec((bm2, n), lambda l, i: (i, 0)),
            pl.BlockSpec((n, nhidden), lambda l, i: (0, 0)),
            pl.BlockSpec((n, nhidden), lambda l, i: (0, 0)),
            pl.BlockSpec((1, nhidden, nhidden), lambda l, i: (l + 1, 0, 0)),
            pl.BlockSpec((nhidden, nclass), lambda l, i: (0, 0)),
            pl.BlockSpec((1, nclass), lambda l, i: (0, 0)),
        ],
        out_specs=pl.BlockSpec((bm2, nclass), lambda l, i: (i, 0)),
        out_shape=jax.ShapeDtypeStruct((n, nclass), jnp.float32),
        scratch_shapes=[
            pltpu.VMEM((2, n, nhidden), jnp.bfloat16),
        ],
    )(gbf, h1, h0, Wc, W2, b2.reshape(1, -1))
    return out
